# baseline (speedup 1.0000x reference)
import functools

import jax
import jax.numpy as jnp
from jax import lax
from jax.experimental import pallas as pl
from jax.experimental.pallas import tpu as pltpu
from jax.experimental.pallas import tpu_sc as plsc

N = 10000
E = 320000
DH = 128
DA = 128
DB = 16
L = 3
EPS = 1e-5

NC = 2
NS = 16
NW = NC * NS
CK = 128
CH = 80
PT = CH * CK
EP = NW * PT
NP = 10240
RPS = NP // NS
EB = 2560


def _zero_vmem(ref, rows, cols):
    per_row = cols // 16

    def body(i, _):
        r = i // per_row
        c = (i % per_row) * 16
        ref[r, pl.ds(c, 16)] = jnp.zeros((16,), jnp.float32)
        return 0

    lax.fori_loop(0, rows * per_row, body, 0)


def _sc_scatter_body(d, rows_hbm, idx_hbm, dst_hbm, out_hbm,
                     idx_v, dst_v, rows_v, rows_b, acc, sem, *, gather):
    cid = lax.axis_index("c")
    sid = lax.axis_index("s")
    wid = cid * NS + sid

    _zero_vmem(rows_v, CK, d)
    for z in range(RPS // CK):
        pltpu.sync_copy(rows_v, acc.at[pl.ds(sid * RPS + z * CK, CK)])
    plsc.subcore_barrier()

    pltpu.sync_copy(dst_hbm.at[pl.ds(wid * CH, CH)], dst_v)

    bufs = (rows_v, rows_b)

    if gather:
        s2 = (idx_v.at[0], idx_v.at[1])
        pltpu.sync_copy(idx_hbm.at[wid * CH], s2[0])
        pltpu.async_copy(rows_hbm.at[s2[0]], bufs[0], sem)
        pltpu.sync_copy(idx_hbm.at[wid * CH + 1], s2[1])

        def pair(g, _):
            for b in range(2):
                j = 2 * g + b
                pltpu.make_async_copy(rows_hbm.at[s2[b]], bufs[b],
                                      sem).wait()

                @pl.when(j + 1 < CH)
                def _():
                    pltpu.async_copy(rows_hbm.at[s2[1 - b]], bufs[1 - b],
                                     sem)

                @pl.when(j + 2 < CH)
                def _():
                    pltpu.sync_copy(
                        idx_hbm.at[wid * CH + jnp.minimum(j + 2, CH - 1)],
                        s2[b])

                pltpu.sync_copy(bufs[b], acc.at[dst_v.at[j]], add=True)
            return 0

        lax.fori_loop(0, CH // 2, pair, 0)
    else:
        def src_at(j):
            return rows_hbm.at[pl.ds(wid * PT + j * CK, CK)]

        pltpu.async_copy(src_at(0), bufs[0], sem)

        def pair(g, _):
            for b in range(2):
                j = 2 * g + b
                pltpu.make_async_copy(src_at(j), bufs[b], sem).wait()

                @pl.when(j + 1 < CH)
                def _():
                    pltpu.async_copy(src_at(jnp.minimum(j + 1, CH - 1)),
                                     bufs[1 - b], sem)

                pltpu.sync_copy(bufs[b], acc.at[dst_v.at[j]], add=True)
            return 0

        lax.fori_loop(0, CH // 2, pair, 0)

    plsc.subcore_barrier()

    pltpu.sync_copy(acc.at[pl.ds(sid * RPS, RPS)],
                    out_hbm.at[cid, pl.ds(sid * RPS, RPS)])


def _make_sc_scatter(d, gather):
    @functools.partial(
        pl.kernel,
        out_type=jax.ShapeDtypeStruct((NC, NP, d), jnp.float32),
        mesh=plsc.VectorSubcoreMesh(core_axis_name="c", subcore_axis_name="s"),
        scratch_types=[
            pltpu.VMEM((2, CK), jnp.int32),
            pltpu.VMEM((CH, CK), jnp.int32),
            pltpu.VMEM((CK, d), jnp.float32),
            pltpu.VMEM((CK, d), jnp.float32),
            pltpu.VMEM_SHARED((NP, d), jnp.float32),
            pltpu.SemaphoreType.DMA,
        ],
    )
    def k(rows_hbm, idx_hbm, dst_hbm, out_hbm,
          idx_v, dst_v, rows_v, rows_b, acc, sem):
        _sc_scatter_body(d, rows_hbm, idx_hbm, dst_hbm, out_hbm,
                         idx_v, dst_v, rows_v, rows_b, acc, sem,
                         gather=gather)

    return k


@functools.lru_cache(maxsize=None)
def _sc_kernels():
    return _make_sc_scatter(DH, gather=False), _make_sc_scatter(DH, gather=True)


def _ea_builder_body(ea_ref, out_ref):
    blk = ea_ref.shape[0]
    out_ref[...] = jnp.concatenate(
        [ea_ref[...],
         jnp.ones((blk, 1), jnp.float32),
         jnp.zeros((blk, DH - DB - 1), jnp.float32)], axis=1)


def _ea_builder(edge_attr):
    return pl.pallas_call(
        _ea_builder_body,
        grid=(EP // EB,),
        in_specs=[pl.BlockSpec((EB, DB),
                               lambda i: (jnp.minimum(i, E // EB - 1), 0))],
        out_specs=pl.BlockSpec((EB, DH), lambda i: (i, 0)),
        out_shape=jax.ShapeDtypeStruct((EP, DH), jnp.float32),
    )(edge_attr)


def _bn(r, g, b):
    m = jnp.mean(r, axis=0, keepdims=True)
    c = r - m
    v = jnp.mean(c * c, axis=0, keepdims=True)
    return c * lax.rsqrt(v + EPS) * g + b


def _tc0_body(p_ref, x_ref, aW_ref, ab_ref, bW_ref, bb_ref, g_ref, b_ref,
              h_ref, ac_ref):
    ac = p_ref[0, :N, :32] + p_ref[1, :N, :32]
    a = ac[:, :DB]
    cnt = ac[:, DB:DB + 1]
    pre = (jnp.dot(x_ref[...], aW_ref[...], preferred_element_type=jnp.float32)
           + ab_ref[...]
           + jnp.dot(a, bW_ref[...], preferred_element_type=jnp.float32)
           + cnt * bb_ref[...])
    r = jnp.maximum(pre, 0.0)
    h_ref[...] = _bn(r, g_ref[...], b_ref[...])
    ac_ref[...] = ac


_tc0 = pl.pallas_call(
    _tc0_body,
    out_shape=[jax.ShapeDtypeStruct((N, DH), jnp.float32),
               jax.ShapeDtypeStruct((N, 32), jnp.float32)],
)


def _tcl_body(p_ref, ac_ref, h_ref, bW_ref, bb_ref, W1_ref, b1_ref,
              W2_ref, b2_ref, g1_ref, n1_ref, g2_ref, n2_ref, out_ref):
    h1 = p_ref[0, :N] + p_ref[1, :N]
    h2 = (jnp.dot(ac_ref[:, :DB], bW_ref[...],
                  preferred_element_type=jnp.float32)
          + ac_ref[:, DB:DB + 1] * bb_ref[...])
    t = jnp.maximum(
        jnp.dot(h1, W1_ref[...], preferred_element_type=jnp.float32)
        + b1_ref[...] + h2, 0.0)
    t = _bn(t, g1_ref[...], n1_ref[...])
    hn = jnp.maximum(
        jnp.dot(t, W2_ref[...], preferred_element_type=jnp.float32)
        + b2_ref[...] + h_ref[...], 0.0)
    out_ref[...] = _bn(hn, g2_ref[...], n2_ref[...])


_tcl = pl.pallas_call(
    _tcl_body,
    out_shape=jax.ShapeDtypeStruct((N, DH), jnp.float32),
)


def _row(v):
    return v.reshape(1, -1)


def kernel(x, edge_index, edge_attr, params):
    pad = EP - E
    srcp = jnp.concatenate(
        [edge_index[0], jnp.zeros((pad,), jnp.int32)]).reshape(EP // CK, CK)
    dstp = jnp.concatenate(
        [edge_index[1],
         jnp.full((pad,), NP - 1, jnp.int32)]).reshape(EP // CK, CK)

    ea_tab = _ea_builder(edge_attr)

    sc_ea, sc_spmm = _sc_kernels()
    p_ea = sc_ea(ea_tab, srcp, dstp)
    h, ac = _tc0(p_ea, x,
                 params['atom_W'], _row(params['atom_b']),
                 params['bond0_W'], _row(params['bond0_b']),
                 _row(params['bn0_g']), _row(params['bn0_b']))
    for lp in params['layers']:
        p = sc_spmm(h, srcp, dstp)
        h = _tcl(p, ac, h,
                 lp['bond_W'], _row(lp['bond_b']),
                 lp['W1'], _row(lp['b1']),
                 lp['W2'], _row(lp['b2']),
                 lp['bn1_g'], _row(lp['bn1_b']),
                 lp['bn2_g'], _row(lp['bn2_b']))
    return h


# speedup vs baseline: 2.1471x; 2.1471x over previous
import functools

import jax
import jax.numpy as jnp
from jax import lax
from jax.experimental import pallas as pl
from jax.experimental.pallas import tpu as pltpu
from jax.experimental.pallas import tpu_sc as plsc

N = 10000
E = 320000
DH = 128
DA = 128
DB = 16
L = 3
EPS = 1e-5

NC = 2
NS = 16
NW = NC * NS
CK = 128
CH = 80
PT = CH * CK
EP = NW * PT
NP = 10240
RPS = NP // NS
EB = 2560


def _zero_vmem(ref, rows, cols):
    per_row = cols // 16

    def body(i, _):
        r = i // per_row
        c = (i % per_row) * 16
        ref[r, pl.ds(c, 16)] = jnp.zeros((16,), jnp.float32)
        return 0

    lax.fori_loop(0, rows * per_row, body, 0)


def _sc_scatter_body(d, rows_hbm, idx_hbm, dst_hbm, out_hbm,
                     idx_v, dst_v, rows_v, rows_b, acc, sem, *, gather):
    cid = lax.axis_index("c")
    sid = lax.axis_index("s")
    wid = cid * NS + sid

    _zero_vmem(rows_v, CK, d)
    for z in range(RPS // CK):
        pltpu.sync_copy(rows_v, acc.at[pl.ds(sid * RPS + z * CK, CK)])
    plsc.subcore_barrier()

    pltpu.sync_copy(dst_hbm.at[pl.ds(wid * CH, CH)], dst_v)

    bufs = (rows_v, rows_b)

    if gather:
        s2 = (idx_v.at[0], idx_v.at[1])
        pltpu.sync_copy(idx_hbm.at[wid * CH], s2[0])
        pltpu.async_copy(rows_hbm.at[s2[0]], bufs[0], sem)
        pltpu.sync_copy(idx_hbm.at[wid * CH + 1], s2[1])

        def pair(g, _):
            for b in range(2):
                j = 2 * g + b
                pltpu.make_async_copy(rows_hbm.at[s2[b]], bufs[b],
                                      sem).wait()

                @pl.when(j + 1 < CH)
                def _():
                    pltpu.async_copy(rows_hbm.at[s2[1 - b]], bufs[1 - b],
                                     sem)

                @pl.when(j + 2 < CH)
                def _():
                    pltpu.sync_copy(
                        idx_hbm.at[wid * CH + jnp.minimum(j + 2, CH - 1)],
                        s2[b])

                pltpu.sync_copy(bufs[b], acc.at[dst_v.at[j]], add=True)
            return 0

        lax.fori_loop(0, CH // 2, pair, 0)
    else:
        def src_at(j):
            return rows_hbm.at[pl.ds(wid * PT + j * CK, CK)]

        pltpu.async_copy(src_at(0), bufs[0], sem)

        def pair(g, _):
            for b in range(2):
                j = 2 * g + b
                pltpu.make_async_copy(src_at(j), bufs[b], sem).wait()

                @pl.when(j + 1 < CH)
                def _():
                    pltpu.async_copy(src_at(jnp.minimum(j + 1, CH - 1)),
                                     bufs[1 - b], sem)

                pltpu.sync_copy(bufs[b], acc.at[dst_v.at[j]], add=True)
            return 0

        lax.fori_loop(0, CH // 2, pair, 0)

    plsc.subcore_barrier()

    pltpu.sync_copy(acc.at[pl.ds(sid * RPS, RPS)],
                    out_hbm.at[cid, pl.ds(sid * RPS, RPS)])


def _make_sc_scatter(d, gather):
    @functools.partial(
        pl.kernel,
        out_type=jax.ShapeDtypeStruct((NC, NP, d), jnp.float32),
        mesh=plsc.VectorSubcoreMesh(core_axis_name="c", subcore_axis_name="s"),
        scratch_types=[
            pltpu.VMEM((2, CK), jnp.int32),
            pltpu.VMEM((CH, CK), jnp.int32),
            pltpu.VMEM((CK, d), jnp.float32),
            pltpu.VMEM((CK, d), jnp.float32),
            pltpu.VMEM_SHARED((NP, d), jnp.float32),
            pltpu.SemaphoreType.DMA,
        ],
    )
    def k(rows_hbm, idx_hbm, dst_hbm, out_hbm,
          idx_v, dst_v, rows_v, rows_b, acc, sem):
        _sc_scatter_body(d, rows_hbm, idx_hbm, dst_hbm, out_hbm,
                         idx_v, dst_v, rows_v, rows_b, acc, sem,
                         gather=gather)

    return k


@functools.lru_cache(maxsize=None)
def _sc_kernels():
    return _make_sc_scatter(DH, gather=False), _make_sc_scatter(DH, gather=True)


def _ea_builder_body(ea_ref, out_ref):
    blk = ea_ref.shape[0]
    out_ref[...] = jnp.concatenate(
        [ea_ref[...],
         jnp.ones((blk, 1), jnp.float32),
         jnp.zeros((blk, DH - DB - 1), jnp.float32)], axis=1)


def _ea_builder(edge_attr):
    return pl.pallas_call(
        _ea_builder_body,
        grid=(EP // EB,),
        in_specs=[pl.BlockSpec((EB, DB),
                               lambda i: (jnp.minimum(i, E // EB - 1), 0))],
        out_specs=pl.BlockSpec((EB, DH), lambda i: (i, 0)),
        out_shape=jax.ShapeDtypeStruct((EP, DH), jnp.float32),
    )(edge_attr)


def _bn(r, g, b):
    m = jnp.mean(r, axis=0, keepdims=True)
    c = r - m
    v = jnp.mean(c * c, axis=0, keepdims=True)
    return c * lax.rsqrt(v + EPS) * g + b


def _tc0_body(p_ref, x_ref, aW_ref, ab_ref, bW_ref, bb_ref, g_ref, b_ref,
              h_ref, ac_ref):
    ac = p_ref[0, :N, :32] + p_ref[1, :N, :32]
    a = ac[:, :DB]
    cnt = ac[:, DB:DB + 1]
    pre = (jnp.dot(x_ref[...], aW_ref[...], preferred_element_type=jnp.float32)
           + ab_ref[...]
           + jnp.dot(a, bW_ref[...], preferred_element_type=jnp.float32)
           + cnt * bb_ref[...])
    r = jnp.maximum(pre, 0.0)
    h_ref[...] = _bn(r, g_ref[...], b_ref[...])
    ac_ref[...] = ac


_tc0 = pl.pallas_call(
    _tc0_body,
    out_shape=[jax.ShapeDtypeStruct((N, DH), jnp.float32),
               jax.ShapeDtypeStruct((N, 32), jnp.float32)],
)


def _tcl_body(p_ref, ac_ref, h_ref, bW_ref, bb_ref, W1_ref, b1_ref,
              W2_ref, b2_ref, g1_ref, n1_ref, g2_ref, n2_ref, out_ref):
    h1 = p_ref[0, :N] + p_ref[1, :N]
    h2 = (jnp.dot(ac_ref[:, :DB], bW_ref[...],
                  preferred_element_type=jnp.float32)
          + ac_ref[:, DB:DB + 1] * bb_ref[...])
    t = jnp.maximum(
        jnp.dot(h1, W1_ref[...], preferred_element_type=jnp.float32)
        + b1_ref[...] + h2, 0.0)
    t = _bn(t, g1_ref[...], n1_ref[...])
    hn = jnp.maximum(
        jnp.dot(t, W2_ref[...], preferred_element_type=jnp.float32)
        + b2_ref[...] + h_ref[...], 0.0)
    out_ref[...] = _bn(hn, g2_ref[...], n2_ref[...])


_tcl = pl.pallas_call(
    _tcl_body,
    out_shape=jax.ShapeDtypeStruct((N, DH), jnp.float32),
)


def _row(v):
    return v.reshape(1, -1)


def kernel(x, edge_index, edge_attr, params):
    pad = EP - E
    fk = jnp.arange(pad, dtype=jnp.int32)
    srcp = jnp.concatenate(
        [edge_index[0], fk % N]).reshape(EP // CK, CK)
    dstp = jnp.concatenate(
        [edge_index[1], N + fk % (NP - N)]).reshape(EP // CK, CK)

    ea_tab = _ea_builder(edge_attr)

    sc_ea, sc_spmm = _sc_kernels()
    p_ea = sc_ea(ea_tab, srcp, dstp)
    h, ac = _tc0(p_ea, x,
                 params['atom_W'], _row(params['atom_b']),
                 params['bond0_W'], _row(params['bond0_b']),
                 _row(params['bn0_g']), _row(params['bn0_b']))
    for lp in params['layers']:
        p = sc_spmm(h, srcp, dstp)
        h = _tcl(p, ac, h,
                 lp['bond_W'], _row(lp['bond_b']),
                 lp['W1'], _row(lp['b1']),
                 lp['W2'], _row(lp['b2']),
                 lp['bn1_g'], _row(lp['bn1_b']),
                 lp['bn2_g'], _row(lp['bn2_b']))
    return h


# speedup vs baseline: 2.5242x; 1.1756x over previous
import functools

import jax
import jax.numpy as jnp
from jax import lax
from jax.experimental import pallas as pl
from jax.experimental.pallas import tpu as pltpu
from jax.experimental.pallas import tpu_sc as plsc

N = 10000
E = 320000
DH = 128
DA = 128
DB = 16
L = 3
EPS = 1e-5

NC = 2
NS = 16
NW = NC * NS
PT = E // NW
CK = 125
CH = PT // CK
NP = 10240
RPS = NP // NS
ZR = 128

def _zero_vmem(ref, rows, cols):
    per_row = cols // 16

    def body(i, _):
        r = i // per_row
        c = (i % per_row) * 16
        ref[r, pl.ds(c, 16)] = jnp.zeros((16,), jnp.float32)
        return 0

    lax.fori_loop(0, rows * per_row, body, 0)


def _sc_scatter_body(d, rows_hbm, idx_hbm, dst_hbm, out_hbm,
                     idx_v, dst_v, rows_v, rows_b, acc, sem, *, gather):
    cid = lax.axis_index("c")
    sid = lax.axis_index("s")
    wid = cid * NS + sid

    _zero_vmem(rows_v, ZR, d)
    for z in range(RPS // ZR):
        pltpu.sync_copy(rows_v, acc.at[pl.ds(sid * RPS + z * ZR, ZR)])
    plsc.subcore_barrier()

    pltpu.sync_copy(dst_hbm.at[wid], dst_v)

    bufs = (rows_v.at[pl.ds(0, CK)], rows_b)
    s2 = (idx_v.at[0], idx_v.at[1])
    pltpu.sync_copy(idx_hbm.at[wid, 0], s2[0])
    pltpu.async_copy(rows_hbm.at[s2[0]], bufs[0], sem)
    pltpu.sync_copy(idx_hbm.at[wid, 1], s2[1])

    def pair(g, _):
        for b in range(2):
            j = 2 * g + b
            pltpu.make_async_copy(rows_hbm.at[s2[b]], bufs[b], sem).wait()

            @pl.when(j + 1 < CH)
            def _():
                pltpu.async_copy(rows_hbm.at[s2[1 - b]], bufs[1 - b], sem)

            @pl.when(j + 2 < CH)
            def _():
                pltpu.sync_copy(
                    idx_hbm.at[wid, jnp.minimum(j + 2, CH - 1)], s2[b])

            pltpu.sync_copy(bufs[b], acc.at[dst_v.at[j]], add=True)
        return 0

    lax.fori_loop(0, CH // 2, pair, 0)
    plsc.subcore_barrier()

    pltpu.sync_copy(acc.at[pl.ds(sid * RPS, RPS)],
                    out_hbm.at[cid, pl.ds(sid * RPS, RPS)])


def _make_sc_scatter(d, gather):
    @functools.partial(
        pl.kernel,
        out_type=jax.ShapeDtypeStruct((NC, NP, d), jnp.float32),
        mesh=plsc.VectorSubcoreMesh(core_axis_name="c", subcore_axis_name="s"),
        scratch_types=[
            pltpu.VMEM((2, CK), jnp.int32),
            pltpu.VMEM((CH, CK), jnp.int32),
            pltpu.VMEM((ZR, d), jnp.float32),
            pltpu.VMEM((CK, d), jnp.float32),
            pltpu.VMEM_SHARED((NP, d), jnp.float32),
            pltpu.SemaphoreType.DMA,
        ],
    )
    def k(rows_hbm, idx_hbm, dst_hbm, out_hbm,
          idx_v, dst_v, rows_v, rows_b, acc, sem):
        _sc_scatter_body(d, rows_hbm, idx_hbm, dst_hbm, out_hbm,
                         idx_v, dst_v, rows_v, rows_b, acc, sem,
                         gather=gather)

    return k


@functools.lru_cache(maxsize=None)
def _sc_kernels():
    return _make_sc_scatter(DH, gather=True)


def _bn(r, g, b):
    m = jnp.mean(r, axis=0, keepdims=True)
    v = jnp.mean(r * r, axis=0, keepdims=True) - m * m
    return (r - m) * lax.rsqrt(v + EPS) * g + b


def _tc0_body(p_ref, x_ref, aW_ref, ab_ref, bW_ref, bb_ref, g_ref, b_ref,
              h_ref, ac_ref):
    ac = p_ref[0, :N, :32] + p_ref[1, :N, :32]
    a = ac[:, :DB]
    cnt = ac[:, DB:DB + 1]
    pre = (jnp.dot(x_ref[...], aW_ref[...], preferred_element_type=jnp.float32)
           + ab_ref[...]
           + jnp.dot(a, bW_ref[...], preferred_element_type=jnp.float32)
           + cnt * bb_ref[...])
    r = jnp.maximum(pre, 0.0)
    h_ref[...] = _bn(r, g_ref[...], b_ref[...])
    ac_ref[...] = ac


_tc0 = pl.pallas_call(
    _tc0_body,
    out_shape=[jax.ShapeDtypeStruct((N, DH), jnp.float32),
               jax.ShapeDtypeStruct((N, 32), jnp.float32)],
)


def _tcl_body(p_ref, ac_ref, h_ref, bW_ref, bb_ref, W1_ref, b1_ref,
              W2_ref, b2_ref, g1_ref, n1_ref, g2_ref, n2_ref, out_ref):
    h1 = p_ref[0, :N] + p_ref[1, :N]
    h2 = (jnp.dot(ac_ref[:, :DB], bW_ref[...],
                  preferred_element_type=jnp.float32)
          + ac_ref[:, DB:DB + 1] * bb_ref[...])
    t = jnp.maximum(
        jnp.dot(h1, W1_ref[...], preferred_element_type=jnp.float32)
        + b1_ref[...] + h2, 0.0)
    t = _bn(t, g1_ref[...], n1_ref[...])
    hn = jnp.maximum(
        jnp.dot(t, W2_ref[...], preferred_element_type=jnp.float32)
        + b2_ref[...] + h_ref[...], 0.0)
    out_ref[...] = _bn(hn, g2_ref[...], n2_ref[...])


_tcl = pl.pallas_call(
    _tcl_body,
    out_shape=jax.ShapeDtypeStruct((N, DH), jnp.float32),
)


def _row(v):
    return v.reshape(1, -1)


def kernel(x, edge_index, edge_attr, params):
    src = edge_index[0].reshape(NW, CH, CK)
    dst = edge_index[1].reshape(NW, CH, CK)
    ea_pad = jnp.concatenate(
        [edge_attr,
         jnp.ones((E, 1), jnp.float32),
         jnp.zeros((E, DH - DB - 1), jnp.float32)], axis=1)
    eid = jnp.arange(E, dtype=jnp.int32).reshape(NW, CH, CK)

    sc_spmm = _sc_kernels()
    p_ea = sc_spmm(ea_pad, eid, dst)
    h, ac = _tc0(p_ea, x,
                 params['atom_W'], _row(params['atom_b']),
                 params['bond0_W'], _row(params['bond0_b']),
                 _row(params['bn0_g']), _row(params['bn0_b']))
    for lp in params['layers']:
        p = sc_spmm(h, src, dst)
        h = _tcl(p, ac, h,
                 lp['bond_W'], _row(lp['bond_b']),
                 lp['W1'], _row(lp['b1']),
                 lp['W2'], _row(lp['b2']),
                 lp['bn1_g'], _row(lp['bn1_b']),
                 lp['bn2_g'], _row(lp['bn2_b']))
    return h
